# baseline (speedup 1.0000x reference)
import functools

import jax
import jax.numpy as jnp
from jax import lax
from jax.experimental import pallas as pl
from jax.experimental.pallas import tpu as pltpu
from jax.experimental.pallas import tpu_sc as plsc

SEQ = 200
D = 64
LANES = 16
NC, NS = 2, 16
NW = NC * NS
HALF = 100


def _body(tok_hbm, word_hbm, pos_hbm, out_hbm, idx_v, pos_v, rows_v, sem):
    wid = lax.axis_index("s") * NC + lax.axis_index("c")
    n_steps = idx_v.shape[0]
    n_chunks = n_steps // 2

    pltpu.sync_copy(tok_hbm.at[wid], idx_v)
    pltpu.sync_copy(pos_hbm, pos_v)

    def chunk(j, carry):
        cp0 = pltpu.async_copy(
            word_hbm.at[idx_v.at[2 * j]], rows_v.at[pl.ds(0, HALF)], sem)
        cp1 = pltpu.async_copy(
            word_hbm.at[idx_v.at[2 * j + 1]], rows_v.at[pl.ds(HALF, HALF)], sem)
        cp0.wait()
        cp1.wait()

        def add_row(r, c):
            for k in range(D // LANES):
                sl = pl.ds(k * LANES, LANES)
                rows_v[r, sl] = rows_v[r, sl] + pos_v[r, sl]
            return c

        lax.fori_loop(0, SEQ, add_row, 0)
        pltpu.sync_copy(rows_v, out_hbm.at[wid, j])
        return carry

    lax.fori_loop(0, n_chunks, chunk, 0)


def kernel(tokens, word_table, pos_table):
    b, s = tokens.shape
    d = word_table.shape[1]
    n = b * s
    n_per_w = n // NW
    n_chunks = n_per_w // SEQ
    tok = tokens.astype(jnp.int32).reshape(NW, 2 * n_chunks, HALF)

    mesh = plsc.VectorSubcoreMesh(core_axis_name="c", subcore_axis_name="s",
                                  num_cores=NC, num_subcores=NS)
    run = pl.kernel(
        _body,
        out_type=jax.ShapeDtypeStruct((NW, n_chunks, SEQ, d), jnp.float32),
        mesh=mesh,
        scratch_types=[
            pltpu.VMEM((2 * n_chunks, HALF), jnp.int32),
            pltpu.VMEM((SEQ, d), jnp.float32),
            pltpu.VMEM((SEQ, d), jnp.float32),
            pltpu.SemaphoreType.DMA,
        ],
        compiler_params=pltpu.CompilerParams(use_tc_tiling_on_sc=False),
    )
    out = run(tok, word_table, pos_table)
    return out.reshape(b, s, d)


# speedup vs baseline: 1.1548x; 1.1548x over previous
import jax
import jax.numpy as jnp
from jax import lax
from jax.experimental import pallas as pl
from jax.experimental.pallas import tpu as pltpu
from jax.experimental.pallas import tpu_sc as plsc

SEQ = 200
D = 64
LANES = 16
NC, NS = 2, 16
NW = NC * NS
HALF = 100
RB = 4


def _body(tok_hbm, word_hbm, pos_hbm, out_hbm, idx_v, pos_v,
          r0, r1, r2, r3, g0, g1, g2, g3, w0, w1, w2, w3):
    rows = [r0, r1, r2, r3]
    sg = [g0, g1, g2, g3]
    sw = [w0, w1, w2, w3]
    wid = lax.axis_index("s") * NC + lax.axis_index("c")
    n_chunks = idx_v.shape[0] // 2

    pltpu.sync_copy(tok_hbm.at[wid], idx_v)
    pltpu.sync_copy(pos_hbm, pos_v)

    dummy = word_hbm.at[pl.ds(0, SEQ)]

    def issue_gather(g, b):
        pltpu.async_copy(
            word_hbm.at[idx_v.at[2 * g]], rows[b].at[pl.ds(0, HALF)], sg[b])
        pltpu.async_copy(
            word_hbm.at[idx_v.at[2 * g + 1]], rows[b].at[pl.ds(HALF, HALF)],
            sg[b])

    issue_gather(0, 0)

    def group(p, carry):
        for b in range(RB):
            g = RB * p + b
            bn = (b + 1) % RB

            @pl.when(g + 1 < n_chunks)
            def _():
                @pl.when(g >= RB - 1)
                def _():
                    pltpu.make_async_copy(rows[bn], out_hbm.at[wid, 0],
                                          sw[bn]).wait()
                issue_gather(g + 1, bn)

            pltpu.make_async_copy(dummy, rows[b], sg[b]).wait()

            def add_rows(r, c):
                for u in range(2):
                    for k in range(D // LANES):
                        sl = pl.ds(k * LANES, LANES)
                        rows[b][2 * r + u, sl] = (rows[b][2 * r + u, sl]
                                                  + pos_v[2 * r + u, sl])
                return c

            lax.fori_loop(0, SEQ // 2, add_rows, 0)
            pltpu.async_copy(rows[b], out_hbm.at[wid, g], sw[b])
        return carry

    lax.fori_loop(0, n_chunks // RB, group, 0)
    for b in range(RB):
        pltpu.make_async_copy(rows[b], out_hbm.at[wid, 0], sw[b]).wait()


def kernel(tokens, word_table, pos_table):
    b, s = tokens.shape
    d = word_table.shape[1]
    n = b * s
    n_per_w = n // NW
    n_chunks = n_per_w // SEQ
    tok = tokens.astype(jnp.int32).reshape(NW, 2 * n_chunks, HALF)

    mesh = plsc.VectorSubcoreMesh(core_axis_name="c", subcore_axis_name="s",
                                  num_cores=NC, num_subcores=NS)
    run = pl.kernel(
        _body,
        out_type=jax.ShapeDtypeStruct((NW, n_chunks, SEQ, d), jnp.float32),
        mesh=mesh,
        scratch_types=(
            [pltpu.VMEM((2 * n_chunks, HALF), jnp.int32),
             pltpu.VMEM((SEQ, d), jnp.float32)]
            + [pltpu.VMEM((SEQ, d), jnp.float32) for _ in range(RB)]
            + [pltpu.SemaphoreType.DMA for _ in range(2 * RB)]
        ),
        compiler_params=pltpu.CompilerParams(use_tc_tiling_on_sc=False),
    )
    out = run(tok, word_table, pos_table)
    return out.reshape(b, s, d)


# speedup vs baseline: 1.1549x; 1.0002x over previous
import jax
import jax.numpy as jnp
from jax import lax
from jax.experimental import pallas as pl
from jax.experimental.pallas import tpu as pltpu
from jax.experimental.pallas import tpu_sc as plsc

SEQ = 200
D = 64
LANES = 16
NC, NS = 2, 16
NW = NC * NS
S0, S1 = 104, 96
RB = 4


def _body(tok_hbm, word_hbm, pos_hbm, out_hbm, idx_v, pos_v,
          r0, r1, r2, r3, g0, g1, g2, g3, w0, w1, w2, w3):
    rows = [r0, r1, r2, r3]
    sg = [g0, g1, g2, g3]
    sw = [w0, w1, w2, w3]
    wid = lax.axis_index("s") * NC + lax.axis_index("c")
    n_chunks = idx_v.shape[0]
    base = wid * n_chunks

    pltpu.sync_copy(tok_hbm.at[pl.ds(base, n_chunks)], idx_v)
    pltpu.sync_copy(pos_hbm, pos_v)

    dummy = word_hbm.at[pl.ds(0, SEQ)]

    def issue_gather(g, b):
        pltpu.async_copy(
            word_hbm.at[idx_v.at[g, pl.ds(0, S0)]],
            rows[b].at[pl.ds(0, S0)], sg[b])
        pltpu.async_copy(
            word_hbm.at[idx_v.at[g, pl.ds(S0, S1)]],
            rows[b].at[pl.ds(S0, S1)], sg[b])

    issue_gather(0, 0)

    def group(p, carry):
        for b in range(RB):
            g = RB * p + b
            bn = (b + 1) % RB

            @pl.when(g + 1 < n_chunks)
            def _():
                @pl.when(g >= RB - 1)
                def _():
                    pltpu.make_async_copy(rows[bn], out_hbm.at[0],
                                          sw[bn]).wait()
                issue_gather(g + 1, bn)

            pltpu.make_async_copy(dummy, rows[b], sg[b]).wait()

            def add_rows(r, c):
                for u in range(2):
                    for k in range(D // LANES):
                        sl = pl.ds(k * LANES, LANES)
                        rows[b][2 * r + u, sl] = (rows[b][2 * r + u, sl]
                                                  + pos_v[2 * r + u, sl])
                return c

            lax.fori_loop(0, SEQ // 2, add_rows, 0)
            pltpu.async_copy(rows[b], out_hbm.at[base + g], sw[b])
        return carry

    lax.fori_loop(0, n_chunks // RB, group, 0)
    for b in range(RB):
        pltpu.make_async_copy(rows[b], out_hbm.at[0], sw[b]).wait()


def kernel(tokens, word_table, pos_table):
    b, s = tokens.shape
    d = word_table.shape[1]
    tok = tokens.astype(jnp.int32)

    mesh = plsc.VectorSubcoreMesh(core_axis_name="c", subcore_axis_name="s",
                                  num_cores=NC, num_subcores=NS)
    run = pl.kernel(
        _body,
        out_type=jax.ShapeDtypeStruct((b, s, d), jnp.float32),
        mesh=mesh,
        scratch_types=(
            [pltpu.VMEM((b // NW, s), jnp.int32),
             pltpu.VMEM((SEQ, d), jnp.float32)]
            + [pltpu.VMEM((SEQ, d), jnp.float32) for _ in range(RB)]
            + [pltpu.SemaphoreType.DMA for _ in range(2 * RB)]
        ),
        compiler_params=pltpu.CompilerParams(use_tc_tiling_on_sc=False),
    )
    return run(tok, word_table, pos_table)
